# Initial kernel scaffold; baseline (speedup 1.0000x reference)
import jax
import jax.numpy as jnp
from jax.experimental import pallas as pl


def kernel(x, W):
    raise NotImplementedError("write your pallas kernel here")


# speedup vs baseline: 1.0620x; 1.0620x over previous
import jax
import jax.numpy as jnp
from jax.experimental import pallas as pl
from jax.experimental.pallas import tpu as pltpu

_D_MODEL = 4096
_N_EXPERTS = 64
_TOP_K = 8
_BT = 512


def _router_body(x_ref, wt_ref, probs_ref, idx_ref, w_ref):
    x = x_ref[...]
    wt = wt_ref[...]
    logits = jnp.dot(x, wt, preferred_element_type=jnp.float32)
    m = jnp.max(logits, axis=-1, keepdims=True)
    e = jnp.exp(logits - m)
    s = jnp.sum(e, axis=-1, keepdims=True)
    probs = e / s
    probs_ref[...] = probs

    iota = jax.lax.broadcasted_iota(jnp.int32, probs.shape, 1)
    cur = probs
    vals = []
    idxs = []
    for _ in range(_TOP_K):
        mv = jnp.max(cur, axis=-1, keepdims=True)
        ik = jnp.min(jnp.where(cur == mv, iota, _N_EXPERTS), axis=-1,
                     keepdims=True)
        vals.append(mv)
        idxs.append(ik)
        cur = jnp.where(iota == ik, -jnp.inf, cur)
    top_vals = jnp.concatenate(vals, axis=-1)
    top_idx = jnp.concatenate(idxs, axis=-1)
    top_vals = top_vals / (jnp.sum(top_vals, axis=-1, keepdims=True) + 1e-9)
    idx_ref[...] = top_idx
    w_ref[...] = top_vals


def kernel(x, W):
    n_tokens = x.shape[0]
    grid = (n_tokens // _BT,)
    wt = W.T
    out_shapes = (
        jax.ShapeDtypeStruct((n_tokens, _N_EXPERTS), jnp.float32),
        jax.ShapeDtypeStruct((n_tokens, _TOP_K), jnp.int32),
        jax.ShapeDtypeStruct((n_tokens, _TOP_K), jnp.float32),
    )
    probs, idx, w = pl.pallas_call(
        _router_body,
        grid=grid,
        in_specs=[
            pl.BlockSpec((_BT, _D_MODEL), lambda i: (i, 0)),
            pl.BlockSpec((_D_MODEL, _N_EXPERTS), lambda i: (0, 0)),
        ],
        out_specs=(
            pl.BlockSpec((_BT, _N_EXPERTS), lambda i: (i, 0)),
            pl.BlockSpec((_BT, _TOP_K), lambda i: (i, 0)),
            pl.BlockSpec((_BT, _TOP_K), lambda i: (i, 0)),
        ),
        out_shape=out_shapes,
        compiler_params=pltpu.CompilerParams(
            dimension_semantics=("arbitrary",),
        ),
    )(x, wt)
    return (idx, w, probs)


# speedup vs baseline: 1.1965x; 1.1266x over previous
import jax
import jax.numpy as jnp
from jax.experimental import pallas as pl
from jax.experimental.pallas import tpu as pltpu

_D_MODEL = 4096
_N_EXPERTS = 64
_TOP_K = 8
_BT = 1024


def _router_body(x_ref, wt_ref, probs_ref, idx_ref, w_ref):
    x = x_ref[...]
    wt = wt_ref[...]
    logits = jnp.dot(x, wt, preferred_element_type=jnp.float32)
    m = jnp.max(logits, axis=-1, keepdims=True)
    e = jnp.exp(logits - m)
    s = jnp.sum(e, axis=-1, keepdims=True)
    probs = e / s
    probs_ref[...] = probs

    iota = jax.lax.broadcasted_iota(jnp.int32, probs.shape, 1)
    cur = probs
    vals = []
    idxs = []
    for _ in range(_TOP_K):
        mv = jnp.max(cur, axis=-1, keepdims=True)
        ik = jnp.min(jnp.where(cur == mv, iota, _N_EXPERTS), axis=-1,
                     keepdims=True)
        vals.append(mv)
        idxs.append(ik)
        cur = jnp.where(iota == ik, -jnp.inf, cur)
    top_vals = jnp.concatenate(vals, axis=-1)
    top_idx = jnp.concatenate(idxs, axis=-1)
    top_vals = top_vals / (jnp.sum(top_vals, axis=-1, keepdims=True) + 1e-9)
    idx_ref[...] = top_idx
    w_ref[...] = top_vals


def kernel(x, W):
    n_tokens = x.shape[0]
    grid = (n_tokens // _BT,)
    wt = W.T
    out_shapes = (
        jax.ShapeDtypeStruct((n_tokens, _N_EXPERTS), jnp.float32),
        jax.ShapeDtypeStruct((n_tokens, _TOP_K), jnp.int32),
        jax.ShapeDtypeStruct((n_tokens, _TOP_K), jnp.float32),
    )
    probs, idx, w = pl.pallas_call(
        _router_body,
        grid=grid,
        in_specs=[
            pl.BlockSpec((_BT, _D_MODEL), lambda i: (i, 0)),
            pl.BlockSpec((_D_MODEL, _N_EXPERTS), lambda i: (0, 0)),
        ],
        out_specs=(
            pl.BlockSpec((_BT, _N_EXPERTS), lambda i: (i, 0)),
            pl.BlockSpec((_BT, _TOP_K), lambda i: (i, 0)),
            pl.BlockSpec((_BT, _TOP_K), lambda i: (i, 0)),
        ),
        out_shape=out_shapes,
        compiler_params=pltpu.CompilerParams(
            dimension_semantics=("arbitrary",),
        ),
    )(x, wt)
    return (idx, w, probs)
